# baseline (speedup 1.0000x reference)
import functools

import jax
import jax.numpy as jnp
from jax.experimental import pallas as pl
from jax.experimental.pallas import tpu as pltpu


def _cfconv_block(feat_ref, rbf_ref, nbr_ref, w1_ref, b1_ref, w2_ref, b2_ref,
                  wf_ref, agg_ref, att_ref, *, TN, K, G, F, N):
    R = TN * K
    f32 = jnp.float32
    feats = feat_ref[0]
    rbf = rbf_ref[0].reshape(R, G)
    nrm = jnp.sqrt(jnp.sum(rbf * rbf, axis=-1, keepdims=True))
    rbfn = rbf / (nrm + 1e-8)
    h = jnp.tanh(jnp.dot(rbfn, w1_ref[...], preferred_element_type=f32)
                 + b1_ref[...])
    cf = jnp.dot(h, w2_ref[...], preferred_element_type=f32) + b2_ref[...]

    idx = nbr_ref[0]
    onehot = (jax.lax.broadcasted_iota(jnp.int32, (R, N), 1) == idx)
    nf = jnp.dot(onehot.astype(jnp.bfloat16), feats,
                 preferred_element_type=f32)

    conv = nf * cf
    lgcol = jnp.sum(conv * wf_ref[...], axis=-1, keepdims=True)

    jmodk = jax.lax.broadcasted_iota(jnp.int32, (R, K), 0) % K
    m_sel = (jmodk == jax.lax.broadcasted_iota(jnp.int32, (R, K), 1)).astype(f32)
    s_rows = jax.lax.broadcasted_iota(jnp.int32, (TN, R), 1) // K
    s_sum = (s_rows == jax.lax.broadcasted_iota(jnp.int32, (TN, R), 0)).astype(f32)
    e_rows = jax.lax.broadcasted_iota(jnp.int32, (R, TN), 0) // K
    s_exp = (e_rows == jax.lax.broadcasted_iota(jnp.int32, (R, TN), 1)).astype(f32)

    lg = jnp.dot(s_sum, m_sel * lgcol, preferred_element_type=f32)
    mx = jnp.max(lg, axis=-1, keepdims=True)
    ex = jnp.exp(lg - mx)
    att = ex / jnp.sum(ex, axis=-1, keepdims=True)
    att_ref[0] = att

    attcol = jnp.sum(jnp.dot(s_exp, att, preferred_element_type=f32) * m_sel,
                     axis=-1, keepdims=True)
    agg_ref[0] = jnp.dot(s_sum, conv * attcol, preferred_element_type=f32)


@functools.partial(jax.jit, static_argnames=("TN",))
def _cfconv(features, rbf_expansion, nbr_col, W1T, b1, W2T, b2, wfT, TN=16):
    B, N, F = features.shape
    _, _, K, G = rbf_expansion.shape
    R = TN * K
    grid = (B, N // TN)
    kern = functools.partial(_cfconv_block, TN=TN, K=K, G=G, F=F, N=N)
    agg, att = pl.pallas_call(
        kern,
        grid=grid,
        in_specs=[
            pl.BlockSpec((1, N, F), lambda b, i: (b, 0, 0)),
            pl.BlockSpec((1, TN, K, G), lambda b, i: (b, i, 0, 0)),
            pl.BlockSpec((1, R, 1), lambda b, i: (b, i, 0)),
            pl.BlockSpec((G, F), lambda b, i: (0, 0)),
            pl.BlockSpec((1, F), lambda b, i: (0, 0)),
            pl.BlockSpec((F, F), lambda b, i: (0, 0)),
            pl.BlockSpec((1, F), lambda b, i: (0, 0)),
            pl.BlockSpec((1, F), lambda b, i: (0, 0)),
        ],
        out_specs=[
            pl.BlockSpec((1, TN, F), lambda b, i: (b, i, 0)),
            pl.BlockSpec((1, TN, K), lambda b, i: (b, i, 0)),
        ],
        out_shape=[
            jax.ShapeDtypeStruct((B, N, F), jnp.float32),
            jax.ShapeDtypeStruct((B, N, K), jnp.float32),
        ],
        compiler_params=pltpu.CompilerParams(
            dimension_semantics=("parallel", "arbitrary")),
    )(features, rbf_expansion, nbr_col, W1T, b1, W2T, b2, wfT)
    return agg, att


def kernel(features, rbf_expansion, neighbor_list, W1, b1, W2, b2, nbr_filter):
    B, N, K = neighbor_list.shape
    nbr_col = neighbor_list.astype(jnp.int32).reshape(B, N * K, 1)
    return _cfconv(features.astype(jnp.bfloat16), rbf_expansion, nbr_col,
                   W1.T, b1.reshape(1, -1), W2.T, b2.reshape(1, -1),
                   nbr_filter.T)


# speedup vs baseline: 1.6119x; 1.6119x over previous
import functools

import jax
import jax.numpy as jnp
from jax.experimental import pallas as pl
from jax.experimental.pallas import tpu as pltpu


def _cfconv_block(feat_ref, rbf_ref, nbr_ref, w1_ref, b1_ref, w2_ref, b2_ref,
                  wf_ref, agg_ref, att_ref, *, TN, K, G, F, N):
    R = TN * K
    f32 = jnp.float32
    feats = feat_ref[0]
    rbf = rbf_ref[0].reshape(R, G)
    nrm = jnp.sqrt(jnp.sum(rbf * rbf, axis=-1, keepdims=True))
    rbfn = rbf / (nrm + 1e-8)
    h = jnp.tanh(jnp.dot(rbfn, w1_ref[...], preferred_element_type=f32)
                 + b1_ref[...])
    cf = jnp.dot(h, w2_ref[...], preferred_element_type=f32) + b2_ref[...]

    idx = nbr_ref[0]
    onehot = (jax.lax.broadcasted_iota(jnp.int32, (R, N), 1) == idx)
    nf = jnp.dot(onehot.astype(jnp.bfloat16), feats,
                 preferred_element_type=f32)

    conv = nf * cf
    lgcol = jnp.sum(conv * wf_ref[...], axis=-1, keepdims=True)

    jmodk = jax.lax.broadcasted_iota(jnp.int32, (R, K), 0) % K
    m_sel = (jmodk == jax.lax.broadcasted_iota(jnp.int32, (R, K), 1)).astype(f32)
    s_rows = jax.lax.broadcasted_iota(jnp.int32, (TN, R), 1) // K
    s_sum = (s_rows == jax.lax.broadcasted_iota(jnp.int32, (TN, R), 0)).astype(f32)
    e_rows = jax.lax.broadcasted_iota(jnp.int32, (R, TN), 0) // K
    s_exp = (e_rows == jax.lax.broadcasted_iota(jnp.int32, (R, TN), 1)).astype(f32)

    lg = jnp.dot(s_sum, m_sel * lgcol, preferred_element_type=f32)
    mx = jnp.max(lg, axis=-1, keepdims=True)
    ex = jnp.exp(lg - mx)
    att = ex / jnp.sum(ex, axis=-1, keepdims=True)
    att_ref[0] = att

    attcol = jnp.sum(jnp.dot(s_exp, att, preferred_element_type=f32) * m_sel,
                     axis=-1, keepdims=True)
    agg_ref[0] = jnp.dot(s_sum, conv * attcol, preferred_element_type=f32)


@functools.partial(jax.jit, static_argnames=("TN",))
def _cfconv(features, rbf_expansion, nbr_col, W1T, b1, W2T, b2, wfT, TN=128):
    B, N, F = features.shape
    _, _, K, G = rbf_expansion.shape
    R = TN * K
    grid = (B, N // TN)
    kern = functools.partial(_cfconv_block, TN=TN, K=K, G=G, F=F, N=N)
    agg, att = pl.pallas_call(
        kern,
        grid=grid,
        in_specs=[
            pl.BlockSpec((1, N, F), lambda b, i: (b, 0, 0)),
            pl.BlockSpec((1, TN, K, G), lambda b, i: (b, i, 0, 0)),
            pl.BlockSpec((1, R, 1), lambda b, i: (b, i, 0)),
            pl.BlockSpec((G, F), lambda b, i: (0, 0)),
            pl.BlockSpec((1, F), lambda b, i: (0, 0)),
            pl.BlockSpec((F, F), lambda b, i: (0, 0)),
            pl.BlockSpec((1, F), lambda b, i: (0, 0)),
            pl.BlockSpec((1, F), lambda b, i: (0, 0)),
        ],
        out_specs=[
            pl.BlockSpec((1, TN, F), lambda b, i: (b, i, 0)),
            pl.BlockSpec((1, TN, K), lambda b, i: (b, i, 0)),
        ],
        out_shape=[
            jax.ShapeDtypeStruct((B, N, F), jnp.float32),
            jax.ShapeDtypeStruct((B, N, K), jnp.float32),
        ],
        compiler_params=pltpu.CompilerParams(
            dimension_semantics=("parallel", "arbitrary")),
    )(features, rbf_expansion, nbr_col, W1T, b1, W2T, b2, wfT)
    return agg, att


def kernel(features, rbf_expansion, neighbor_list, W1, b1, W2, b2, nbr_filter):
    B, N, K = neighbor_list.shape
    nbr_col = neighbor_list.astype(jnp.int32).reshape(B, N * K, 1)
    return _cfconv(features.astype(jnp.bfloat16), rbf_expansion, nbr_col,
                   W1.T, b1.reshape(1, -1), W2.T, b2.reshape(1, -1),
                   nbr_filter.T)
